# baseline (speedup 1.0000x reference)
import functools

import jax
import jax.numpy as jnp
from jax import lax
from jax.experimental import pallas as pl
from jax.experimental.pallas import tpu as pltpu
from jax.experimental.pallas import tpu_sc as plsc

N = 100000
M = 50000
D = 128

NW = 32
OWN = N // NW
CH = 128
NCH = 25
MP = 50176
MR = MP // 128

_mesh = plsc.VectorSubcoreMesh(core_axis_name="c", subcore_axis_name="s")
_cp = pltpu.CompilerParams(needs_layout_passes=False)


@functools.partial(
    pl.kernel,
    out_type=jax.ShapeDtypeStruct((N, D), jnp.float32),
    mesh=_mesh,
    scratch_types=[
        pltpu.VMEM((MR, 128), jnp.int32),
        pltpu.VMEM((NCH * CH,), jnp.int32),
        pltpu.VMEM((NCH, CH), jnp.int32),
        pltpu.VMEM((NCH, CH), jnp.int32),
        pltpu.VMEM((2, CH, D), jnp.float32),
        pltpu.VMEM((CH, D), jnp.float32),
        pltpu.SemaphoreType.DMA,
        pltpu.SemaphoreType.DMA,
        pltpu.SemaphoreType.DMA,
        pltpu.SemaphoreType.DMA,
    ],
    compiler_params=_cp,
)
def _unpool_sc(h_hbm, idx_hbm, out_hbm, idxv, winner, dstd, dstz, stage,
               zbuf, semi, semg, semd, semz):
    w = lax.axis_index("s") * 2 + lax.axis_index("c")
    base = w * OWN
    iota = lax.iota(jnp.int32, 16)
    neg1 = jnp.full((16,), -1, jnp.int32)

    cidx = pltpu.async_copy(idx_hbm, idxv, semi)

    def init_winner(ch, c):
        for q in range(8):
            winner[pl.ds(ch * 128 + q * 16, 16)] = neg1
        return c

    lax.fori_loop(0, NCH, init_winner, 0)

    zero16 = jnp.zeros((16,), jnp.float32)

    def init_zbuf(r, c):
        for q in range(8):
            zbuf[r, pl.ds(q * 16, 16)] = zero16
        return c

    lax.fori_loop(0, CH, init_zbuf, 0)

    cidx.wait()

    own_u = jnp.full((16,), OWN, jnp.uint32)

    def scan_body(i, c):
        jbase = jnp.full((16,), i * 256, jnp.int32)
        for r in range(2):
            for q in range(8):
                j16 = idxv[i * 2 + r, pl.ds(q * 16, 16)]
                rel = j16 - base
                cov = lax.bitcast_convert_type(rel, jnp.uint32) < own_u
                vals = jbase + (iota + (r * 128 + q * 16))
                plsc.store_scatter(winner, [rel], vals, mask=cov)
        return c

    lax.fori_loop(0, MR // 2, scan_body, 0)

    def build_lists(ch):
        for q in range(8):
            local = ch * CH + q * 16
            w16 = winner[pl.ds(ch * 128 + q * 16, 16)]
            cov = w16 >= 0
            gr = jnp.full((16,), base + local, jnp.int32) + iota
            if local + 16 <= OWN:
                dstd[ch, pl.ds(q * 16, 16)] = jnp.where(cov, gr, -1)
                dstz[ch, pl.ds(q * 16, 16)] = jnp.where(cov, -1, gr)
            elif local < OWN:
                valid = iota < (OWN - local)
                dstd[ch, pl.ds(q * 16, 16)] = jnp.where(cov & valid, gr, -1)
                dstz[ch, pl.ds(q * 16, 16)] = jnp.where((~cov) & valid, gr, -1)
            else:
                dstd[ch, pl.ds(q * 16, 16)] = neg1
                dstz[ch, pl.ds(q * 16, 16)] = neg1

    def gather(ch, buf):
        return pltpu.async_copy(
            h_hbm.at[plsc.Indices(winner.at[pl.ds(ch * CH, CH)],
                                  ignored_value=-1)],
            stage.at[buf], semg)

    gathers = [None] * NCH
    dscat = [None] * NCH
    zscat = [None] * NCH

    build_lists(0)
    gathers[0] = gather(0, 0)
    for ch in range(NCH):
        buf = ch & 1
        if ch + 1 < NCH:
            build_lists(ch + 1)
            if ch >= 1:
                dscat[ch - 1].wait()
            gathers[ch + 1] = gather(ch + 1, 1 - buf)
        gathers[ch].wait()
        dscat[ch] = pltpu.async_copy(
            stage.at[buf],
            out_hbm.at[plsc.Indices(dstd.at[ch], ignored_value=-1)], semd)
        zscat[ch] = pltpu.async_copy(
            zbuf,
            out_hbm.at[plsc.Indices(dstz.at[ch], ignored_value=-1)], semz)
    dscat[NCH - 1].wait()
    for ch in range(NCH):
        zscat[ch].wait()


_GBLK = 5000


def _copy_body(src_ref, out_ref):
    out_ref[...] = src_ref[...]


def kernel(g, h, pre_h, idx):
    del pre_h
    idx_p = jnp.pad(idx, (0, MP - M), constant_values=-1).reshape(MR, 128)
    new_h = _unpool_sc(h, idx_p)
    g_out = pl.pallas_call(
        _copy_body,
        grid=(N // _GBLK,),
        in_specs=[pl.BlockSpec((_GBLK, D), lambda i: (i, 0))],
        out_specs=pl.BlockSpec((_GBLK, D), lambda i: (i, 0)),
        out_shape=jax.ShapeDtypeStruct((N, D), g.dtype),
    )(g)
    return (g_out, new_h)


# speedup vs baseline: 1.0469x; 1.0469x over previous
import functools

import jax
import jax.numpy as jnp
from jax import lax
from jax.experimental import pallas as pl
from jax.experimental.pallas import tpu as pltpu
from jax.experimental.pallas import tpu_sc as plsc

N = 100000
M = 50000
D = 128

NW = 32
OWN = N // NW
CH = 128
NCH = 25
MP = 50176
MR = MP // 128

_mesh = plsc.VectorSubcoreMesh(core_axis_name="c", subcore_axis_name="s")
_cp = pltpu.CompilerParams(needs_layout_passes=False)


@functools.partial(
    pl.kernel,
    out_type=jax.ShapeDtypeStruct((N, D), jnp.float32),
    mesh=_mesh,
    scratch_types=[
        pltpu.VMEM((MR, 128), jnp.int32),
        pltpu.VMEM((NCH * CH,), jnp.int32),
        pltpu.VMEM((NCH, CH), jnp.int32),
        pltpu.VMEM((NCH, CH), jnp.int32),
        pltpu.VMEM((3, CH, D), jnp.float32),
        pltpu.VMEM((CH, D), jnp.float32),
        pltpu.SemaphoreType.DMA,
        pltpu.SemaphoreType.DMA,
        pltpu.SemaphoreType.DMA,
        pltpu.SemaphoreType.DMA,
    ],
    compiler_params=_cp,
)
def _unpool_sc(h_hbm, idx_hbm, out_hbm, idxv, winner, dstd, dstz, stage,
               zbuf, semi, semg, semd, semz):
    w = lax.axis_index("s") * 2 + lax.axis_index("c")
    base = w * OWN
    iota = lax.iota(jnp.int32, 16)
    neg1 = jnp.full((16,), -1, jnp.int32)

    half = 200
    cidx1 = pltpu.async_copy(idx_hbm.at[pl.ds(0, half)],
                             idxv.at[pl.ds(0, half)], semi)
    cidx2 = pltpu.async_copy(idx_hbm.at[pl.ds(half, MR - half)],
                             idxv.at[pl.ds(half, MR - half)], semi)

    def init_winner(ch, c):
        for q in range(8):
            winner[pl.ds(ch * 128 + q * 16, 16)] = neg1
        return c

    lax.fori_loop(0, NCH, init_winner, 0)

    zero16 = jnp.zeros((16,), jnp.float32)

    def init_zbuf(r, c):
        for q in range(8):
            zbuf[r, pl.ds(q * 16, 16)] = zero16
        return c

    lax.fori_loop(0, CH, init_zbuf, 0)

    own_u = jnp.full((16,), OWN, jnp.uint32)

    def scan_body(i, c):
        jbase = jnp.full((16,), i * 256, jnp.int32)
        for r in range(2):
            for q in range(8):
                j16 = idxv[i * 2 + r, pl.ds(q * 16, 16)]
                rel = j16 - base
                cov = lax.bitcast_convert_type(rel, jnp.uint32) < own_u
                vals = jbase + (iota + (r * 128 + q * 16))
                plsc.store_scatter(winner, [rel], vals, mask=cov)
        return c

    cidx1.wait()
    lax.fori_loop(0, half // 2, scan_body, 0)
    cidx2.wait()
    lax.fori_loop(half // 2, MR // 2, scan_body, 0)

    def build_lists(ch):
        for q in range(8):
            local = ch * CH + q * 16
            w16 = winner[pl.ds(ch * 128 + q * 16, 16)]
            cov = w16 >= 0
            gr = jnp.full((16,), base + local, jnp.int32) + iota
            if local + 16 <= OWN:
                dstd[ch, pl.ds(q * 16, 16)] = jnp.where(cov, gr, -1)
                dstz[ch, pl.ds(q * 16, 16)] = jnp.where(cov, -1, gr)
            elif local < OWN:
                valid = iota < (OWN - local)
                dstd[ch, pl.ds(q * 16, 16)] = jnp.where(cov & valid, gr, -1)
                dstz[ch, pl.ds(q * 16, 16)] = jnp.where((~cov) & valid, gr, -1)
            else:
                dstd[ch, pl.ds(q * 16, 16)] = neg1
                dstz[ch, pl.ds(q * 16, 16)] = neg1

    def gather(ch, buf):
        return pltpu.async_copy(
            h_hbm.at[plsc.Indices(winner.at[pl.ds(ch * CH, CH)],
                                  ignored_value=-1)],
            stage.at[buf], semg)

    gathers = [None] * NCH
    dscat = [None] * NCH
    zscat = [None] * NCH

    build_lists(0)
    build_lists(1)
    gathers[0] = gather(0, 0)
    gathers[1] = gather(1, 1)
    for ch in range(NCH):
        buf = ch % 3
        if ch + 2 < NCH:
            build_lists(ch + 2)
            if ch >= 1:
                dscat[ch - 1].wait()
            gathers[ch + 2] = gather(ch + 2, (ch + 2) % 3)
        gathers[ch].wait()
        dscat[ch] = pltpu.async_copy(
            stage.at[buf],
            out_hbm.at[plsc.Indices(dstd.at[ch], ignored_value=-1)], semd)
        zscat[ch] = pltpu.async_copy(
            zbuf,
            out_hbm.at[plsc.Indices(dstz.at[ch], ignored_value=-1)], semz)
    dscat[NCH - 2].wait()
    dscat[NCH - 1].wait()
    for ch in range(NCH):
        zscat[ch].wait()


_GBLK = 5000


def _copy_body(src_ref, out_ref):
    out_ref[...] = src_ref[...]


def kernel(g, h, pre_h, idx):
    del pre_h
    idx_p = jnp.pad(idx, (0, MP - M), constant_values=-1).reshape(MR, 128)
    new_h = _unpool_sc(h, idx_p)
    g_out = pl.pallas_call(
        _copy_body,
        grid=(N // _GBLK,),
        in_specs=[pl.BlockSpec((_GBLK, D), lambda i: (i, 0))],
        out_specs=pl.BlockSpec((_GBLK, D), lambda i: (i, 0)),
        out_shape=jax.ShapeDtypeStruct((N, D), g.dtype),
    )(g)
    return (g_out, new_h)
